# baseline (speedup 1.0000x reference)
import functools

import jax
import jax.numpy as jnp
from jax.experimental import pallas as pl
from jax.experimental.pallas import tpu as pltpu

T = 4096
D = 2048
E = 8

_ROUTER_BM = 512
_BM = 256
_BN = 2048


def _router_kernel(pred_ref, wp_ref, bp_ref, idx_ref, acc_ref):
    i = pl.program_id(0)

    @pl.when(i == 0)
    def _init():
        acc_ref[...] = jnp.zeros_like(acc_ref)

    part = jnp.dot(pred_ref[...], wp_ref[...],
                   preferred_element_type=jnp.float32)
    acc_ref[...] += jnp.sum(part, axis=0, keepdims=True)

    @pl.when(i == pl.num_programs(0) - 1)
    def _finish():
        scores = acc_ref[...] + jnp.float32(T) * bp_ref[...]
        m = jnp.max(scores)
        lane = jax.lax.broadcasted_iota(jnp.int32, scores.shape, 1)
        idx = jnp.min(jnp.where(scores == m, lane, jnp.int32(2**30)))
        idx_ref[0, 0] = idx


def _dispatch_kernel(idx_ref, x_ref, w_ref, b_ref, o_ref):
    del idx_ref
    x16 = x_ref[...].astype(jnp.bfloat16)
    w16 = w_ref[0].astype(jnp.bfloat16)
    o_ref[...] = (jnp.dot(x16, w16, preferred_element_type=jnp.float32)
                  + b_ref[0])


def kernel(predicate, input, W_pred, b_pred, W_experts, b_experts):
    bp2 = b_pred.reshape(1, E)

    idx = pl.pallas_call(
        _router_kernel,
        grid=(T // _ROUTER_BM,),
        in_specs=[
            pl.BlockSpec((_ROUTER_BM, D), lambda i: (i, 0)),
            pl.BlockSpec((D, E), lambda i: (0, 0)),
            pl.BlockSpec((1, E), lambda i: (0, 0)),
        ],
        out_specs=pl.BlockSpec(memory_space=pltpu.MemorySpace.SMEM),
        out_shape=jax.ShapeDtypeStruct((1, 1), jnp.int32),
        scratch_shapes=[pltpu.VMEM((1, E), jnp.float32)],
    )(predicate, W_pred, bp2)

    idx_flat = idx.reshape((1,))
    be3 = b_experts.reshape(E, 1, D)

    grid_spec = pltpu.PrefetchScalarGridSpec(
        num_scalar_prefetch=1,
        grid=(D // _BN, T // _BM),
        in_specs=[
            pl.BlockSpec((_BM, D), lambda j, i, s: (i, 0)),
            pl.BlockSpec((1, D, _BN), lambda j, i, s: (s[0], 0, j)),
            pl.BlockSpec((1, 1, _BN), lambda j, i, s: (s[0], 0, j)),
        ],
        out_specs=pl.BlockSpec((_BM, _BN), lambda j, i, s: (i, j)),
    )

    out = pl.pallas_call(
        _dispatch_kernel,
        grid_spec=grid_spec,
        out_shape=jax.ShapeDtypeStruct((T, D), jnp.float32),
    )(idx_flat, input, W_experts, be3)
    return out


# speedup vs baseline: 1.0070x; 1.0070x over previous
import functools

import jax
import jax.numpy as jnp
from jax.experimental import pallas as pl
from jax.experimental.pallas import tpu as pltpu

T = 4096
D = 2048
E = 8

_ROUTER_BM = 512
_BM = 1024
_BN = 2048


def _router_kernel(pred_ref, wp_ref, bp_ref, idx_ref, acc_ref):
    i = pl.program_id(0)

    @pl.when(i == 0)
    def _init():
        acc_ref[...] = jnp.zeros_like(acc_ref)

    part = jnp.dot(pred_ref[...], wp_ref[...],
                   preferred_element_type=jnp.float32)
    acc_ref[...] += jnp.sum(part, axis=0, keepdims=True)

    @pl.when(i == pl.num_programs(0) - 1)
    def _finish():
        scores = acc_ref[...] + jnp.float32(T) * bp_ref[...]
        m = jnp.max(scores)
        lane = jax.lax.broadcasted_iota(jnp.int32, scores.shape, 1)
        idx = jnp.min(jnp.where(scores == m, lane, jnp.int32(2**30)))
        idx_ref[0, 0] = idx


def _dispatch_kernel(idx_ref, x_ref, w_ref, b_ref, o_ref):
    del idx_ref
    x16 = x_ref[...].astype(jnp.bfloat16)
    w16 = w_ref[0].astype(jnp.bfloat16)
    o_ref[...] = (jnp.dot(x16, w16, preferred_element_type=jnp.float32)
                  + b_ref[0])


def kernel(predicate, input, W_pred, b_pred, W_experts, b_experts):
    bp2 = b_pred.reshape(1, E)

    idx = pl.pallas_call(
        _router_kernel,
        grid=(T // _ROUTER_BM,),
        in_specs=[
            pl.BlockSpec((_ROUTER_BM, D), lambda i: (i, 0)),
            pl.BlockSpec((D, E), lambda i: (0, 0)),
            pl.BlockSpec((1, E), lambda i: (0, 0)),
        ],
        out_specs=pl.BlockSpec(memory_space=pltpu.MemorySpace.SMEM),
        out_shape=jax.ShapeDtypeStruct((1, 1), jnp.int32),
        scratch_shapes=[pltpu.VMEM((1, E), jnp.float32)],
    )(predicate, W_pred, bp2)

    idx_flat = idx.reshape((1,))
    be3 = b_experts.reshape(E, 1, D)

    grid_spec = pltpu.PrefetchScalarGridSpec(
        num_scalar_prefetch=1,
        grid=(D // _BN, T // _BM),
        in_specs=[
            pl.BlockSpec((_BM, D), lambda j, i, s: (i, 0)),
            pl.BlockSpec((1, D, _BN), lambda j, i, s: (s[0], 0, j)),
            pl.BlockSpec((1, 1, _BN), lambda j, i, s: (s[0], 0, j)),
        ],
        out_specs=pl.BlockSpec((_BM, _BN), lambda j, i, s: (i, j)),
    )

    out = pl.pallas_call(
        _dispatch_kernel,
        grid_spec=grid_spec,
        out_shape=jax.ShapeDtypeStruct((T, D), jnp.float32),
    )(idx_flat, input, W_experts, be3)
    return out


# speedup vs baseline: 1.0351x; 1.0280x over previous
import functools

import jax
import jax.numpy as jnp
from jax.experimental import pallas as pl
from jax.experimental.pallas import tpu as pltpu

T = 4096
D = 2048
E = 8

_ROUTER_BM = 512
_BM = 512
_BN = 2048


def _router_kernel(pred_ref, wp_ref, bp_ref, idx_ref, acc_ref):
    i = pl.program_id(0)

    @pl.when(i == 0)
    def _init():
        acc_ref[...] = jnp.zeros_like(acc_ref)

    part = jnp.dot(pred_ref[...], wp_ref[...],
                   preferred_element_type=jnp.float32)
    acc_ref[...] += jnp.sum(part, axis=0, keepdims=True)

    @pl.when(i == pl.num_programs(0) - 1)
    def _finish():
        scores = acc_ref[...] + jnp.float32(T) * bp_ref[...]
        m = jnp.max(scores)
        lane = jax.lax.broadcasted_iota(jnp.int32, scores.shape, 1)
        idx = jnp.min(jnp.where(scores == m, lane, jnp.int32(2**30)))
        idx_ref[0, 0] = idx


def _dispatch_kernel(idx_ref, x_ref, w_ref, b_ref, o_ref):
    del idx_ref
    o_ref[...] = (jnp.dot(x_ref[...], w_ref[0],
                          preferred_element_type=jnp.float32)
                  + b_ref[0])


def kernel(predicate, input, W_pred, b_pred, W_experts, b_experts):
    bp2 = b_pred.reshape(1, E)

    idx = pl.pallas_call(
        _router_kernel,
        grid=(T // _ROUTER_BM,),
        in_specs=[
            pl.BlockSpec((_ROUTER_BM, D), lambda i: (i, 0)),
            pl.BlockSpec((D, E), lambda i: (0, 0)),
            pl.BlockSpec((1, E), lambda i: (0, 0)),
        ],
        out_specs=pl.BlockSpec(memory_space=pltpu.MemorySpace.SMEM),
        out_shape=jax.ShapeDtypeStruct((1, 1), jnp.int32),
        scratch_shapes=[pltpu.VMEM((1, E), jnp.float32)],
    )(predicate, W_pred, bp2)

    idx_flat = idx.reshape((1,))
    be3 = b_experts.reshape(E, 1, D)

    grid_spec = pltpu.PrefetchScalarGridSpec(
        num_scalar_prefetch=1,
        grid=(D // _BN, T // _BM),
        in_specs=[
            pl.BlockSpec((_BM, D), lambda j, i, s: (i, 0)),
            pl.BlockSpec((1, D, _BN), lambda j, i, s: (s[0], 0, j)),
            pl.BlockSpec((1, 1, _BN), lambda j, i, s: (s[0], 0, j)),
        ],
        out_specs=pl.BlockSpec((_BM, _BN), lambda j, i, s: (i, j)),
    )

    out = pl.pallas_call(
        _dispatch_kernel,
        grid_spec=grid_spec,
        out_shape=jax.ShapeDtypeStruct((T, D), jnp.float32),
    )(idx_flat, input, W_experts, be3)
    return out


# speedup vs baseline: 1.0489x; 1.0133x over previous
import jax
import jax.numpy as jnp
from jax.experimental import pallas as pl
from jax.experimental.pallas import tpu as pltpu

T = 4096
D = 2048
E = 8

_BM = 512
_NB = T // _BM
_STEPS = 2 * _NB


def _fused_kernel(pred_ref, wp_ref, bp_ref, x_ref, w_hbm, b_hbm, o_ref,
                  acc_ref, idx_ref, w_ref, b_ref, w_sem, b_sem):
    i = pl.program_id(0)

    @pl.when(i == 0)
    def _init():
        acc_ref[...] = jnp.zeros_like(acc_ref)

    @pl.when(i < _NB)
    def _router():
        part = jnp.dot(pred_ref[...], wp_ref[...],
                       preferred_element_type=jnp.float32)
        acc_ref[...] += jnp.sum(part, axis=0, keepdims=True)

    @pl.when(i == _NB - 1)
    def _pick_expert():
        scores = acc_ref[...] + jnp.float32(T) * bp_ref[...]
        m = jnp.max(scores)
        lane = jax.lax.broadcasted_iota(jnp.int32, scores.shape, 1)
        idx = jnp.min(jnp.where(scores == m, lane, jnp.int32(2**30)))
        idx_ref[0] = idx
        pltpu.make_async_copy(w_hbm.at[idx], w_ref, w_sem).start()
        pltpu.make_async_copy(b_hbm.at[idx], b_ref, b_sem).start()

    @pl.when(i == _NB)
    def _wait_w():
        pltpu.make_async_copy(w_hbm.at[idx_ref[0]], w_ref, w_sem).wait()
        pltpu.make_async_copy(b_hbm.at[idx_ref[0]], b_ref, b_sem).wait()

    @pl.when(i >= _NB)
    def _dispatch():
        o_ref[...] = (jnp.dot(x_ref[...], w_ref[...],
                              preferred_element_type=jnp.float32)
                      + b_ref[...])


def kernel(predicate, input, W_pred, b_pred, W_experts, b_experts):
    bp2 = b_pred.reshape(1, E)
    be3 = b_experts.reshape(E, 1, D)

    out = pl.pallas_call(
        _fused_kernel,
        grid=(_STEPS,),
        in_specs=[
            pl.BlockSpec((_BM, D), lambda i: (jnp.minimum(i, _NB - 1), 0)),
            pl.BlockSpec((D, E), lambda i: (0, 0)),
            pl.BlockSpec((1, E), lambda i: (0, 0)),
            pl.BlockSpec((_BM, D), lambda i: (jnp.maximum(i - _NB, 0), 0)),
            pl.BlockSpec(memory_space=pltpu.MemorySpace.HBM),
            pl.BlockSpec(memory_space=pltpu.MemorySpace.HBM),
        ],
        out_specs=pl.BlockSpec((_BM, D), lambda i: (jnp.maximum(i - _NB, 0), 0)),
        out_shape=jax.ShapeDtypeStruct((T, D), jnp.float32),
        scratch_shapes=[
            pltpu.VMEM((1, E), jnp.float32),
            pltpu.SMEM((1,), jnp.int32),
            pltpu.VMEM((D, D), jnp.float32),
            pltpu.VMEM((1, D), jnp.float32),
            pltpu.SemaphoreType.DMA,
            pltpu.SemaphoreType.DMA,
        ],
    )(predicate, W_pred, bp2, input, W_experts, be3)
    return out
